# Initial kernel scaffold; baseline (speedup 1.0000x reference)
import functools
import jax
import jax.numpy as jnp
from jax import lax
from jax.experimental import pallas as pl
from jax.experimental.pallas import tpu as pltpu
from jax.experimental.pallas import tpu_sc as plsc

N = 10000
D = 128
E = 320000
NC = 2
NS = 16
NW = NC * NS
CH = 128
EPT = 10112
E_PAD = EPT * NW
NCH = EPT // CH
N2 = 10240
RPT = N // NS
RQ = 125

_mesh = plsc.VectorSubcoreMesh(core_axis_name="c", subcore_axis_name="s")


@functools.partial(
    pl.kernel,
    out_type=jax.ShapeDtypeStruct((NW, N2), jnp.float32),
    mesh=_mesh,
    scratch_types=[
        pltpu.VMEM((N2,), jnp.float32),
        pltpu.VMEM((CH,), jnp.int32),
        pltpu.VMEM((CH,), jnp.float32),
    ],
)
def _deg_kernel(dst_hbm, ew_hbm, parts_hbm, degv, dstv, ewv):
    cid = lax.axis_index("c")
    sid = lax.axis_index("s")
    wid = sid * NC + cid

    def zero(i, _):
        degv[pl.ds(i * 16, 16)] = jnp.zeros((16,), jnp.float32)
        return 0

    lax.fori_loop(0, N2 // 16, zero, 0)

    def chunk(j, _):
        base = wid * EPT + j * CH
        pltpu.sync_copy(dst_hbm.at[pl.ds(base, CH)], dstv)
        pltpu.sync_copy(ew_hbm.at[pl.ds(base, CH)], ewv)
        for g in range(CH // 16):
            idx = dstv[pl.ds(g * 16, 16)]
            w = ewv[pl.ds(g * 16, 16)]
            plsc.addupdate_scatter(degv, [idx], w)
        return 0

    lax.fori_loop(0, NCH, chunk, 0)
    pltpu.sync_copy(degv, parts_hbm.at[wid])


def _msg_body(write_dfull, xw_hbm, dinv_hbm, src_hbm, dst_hbm, ew_hbm,
              p_hbm, dfull_hbm, acc, srcv, dstv, ewv, wbuf, rows, dinv_v,
              zbuf, sem):
    cid = lax.axis_index("c")
    sid = lax.axis_index("s")
    wid = sid * NC + cid

    pltpu.sync_copy(dinv_hbm, dinv_v)

    def zb(i, _):
        for k in range(8):
            zbuf[i, pl.ds(k * 16, 16)] = jnp.zeros((16,), jnp.float32)
        return 0

    lax.fori_loop(0, RQ, zb, 0)
    for q in range(RPT // RQ):
        pltpu.sync_copy(zbuf, acc.at[pl.ds(sid * RPT + q * RQ, RQ)])

    if write_dfull:
        @pl.when(cid == 0)
        def _():
            for q in range(5):
                def fill(r, _):
                    dval = dinv_v[sid * 640 + q * 128 + r]
                    dv = jnp.full((16,), dval, jnp.float32)
                    for k in range(8):
                        rows[r, pl.ds(k * 16, 16)] = dv
                    return 0

                lax.fori_loop(0, 128, fill, 0)
                pltpu.sync_copy(
                    rows, dfull_hbm.at[pl.ds(sid * 640 + q * 128, 128)])

    plsc.subcore_barrier()

    def chunk(j, _):
        base = wid * EPT + j * CH
        pltpu.sync_copy(src_hbm.at[pl.ds(base, CH)], srcv)
        pltpu.sync_copy(dst_hbm.at[pl.ds(base, CH)], dstv)
        pltpu.sync_copy(ew_hbm.at[pl.ds(base, CH)], ewv)
        pltpu.async_copy(xw_hbm.at[srcv], rows, sem).wait()
        for g in range(CH // 16):
            idx = srcv[pl.ds(g * 16, 16)]
            dsv = plsc.load_gather(dinv_v, [idx])
            wbuf[pl.ds(g * 16, 16)] = ewv[pl.ds(g * 16, 16)] * dsv

        def srow(e, _):
            wv = jnp.full((16,), wbuf[e], jnp.float32)
            for k in range(8):
                rows[e, pl.ds(k * 16, 16)] = rows[e, pl.ds(k * 16, 16)] * wv
            return 0

        lax.fori_loop(0, CH, srow, 0)
        pltpu.sync_copy(rows, acc.at[dstv], add=True)
        return 0

    lax.fori_loop(0, NCH, chunk, 0)
    plsc.subcore_barrier()

    for q in range(RPT // RQ):
        r0 = sid * RPT + q * RQ
        pltpu.sync_copy(acc.at[pl.ds(r0, RQ)], p_hbm.at[cid, pl.ds(r0, RQ)])


_msg_scratch = [
    pltpu.VMEM_SHARED((N, D), jnp.float32),
    pltpu.VMEM((CH,), jnp.int32),
    pltpu.VMEM((CH,), jnp.int32),
    pltpu.VMEM((CH,), jnp.float32),
    pltpu.VMEM((CH,), jnp.float32),
    pltpu.VMEM((CH, D), jnp.float32),
    pltpu.VMEM((N2,), jnp.float32),
    pltpu.VMEM((RQ, D), jnp.float32),
    pltpu.SemaphoreType.DMA,
]

_msg_kernel_c1 = functools.partial(
    pl.kernel,
    out_type=(jax.ShapeDtypeStruct((NC, N, D), jnp.float32),
              jax.ShapeDtypeStruct((N2, D), jnp.float32)),
    mesh=_mesh,
    scratch_types=_msg_scratch,
)(functools.partial(_msg_body, True))


def _msg_body2(xw_hbm, dinv_hbm, src_hbm, dst_hbm, ew_hbm, p_hbm,
               acc, srcv, dstv, ewv, wbuf, rows, dinv_v, zbuf, sem):
    _msg_body(False, xw_hbm, dinv_hbm, src_hbm, dst_hbm, ew_hbm,
              p_hbm, None, acc, srcv, dstv, ewv, wbuf, rows, dinv_v,
              zbuf, sem)


_msg_kernel_c2 = functools.partial(
    pl.kernel,
    out_type=jax.ShapeDtypeStruct((NC, N, D), jnp.float32),
    mesh=_mesh,
    scratch_types=_msg_scratch,
)(_msg_body2)


def _tc_b_body(parts_ref, x_ref, w1_ref, dinv_ref, xw_ref):
    deg = 1.0 + jnp.sum(parts_ref[...], axis=0)
    dinv_ref[...] = lax.rsqrt(deg)
    xw_ref[...] = jnp.dot(x_ref[...], w1_ref[...],
                          preferred_element_type=jnp.float32)


def _tc_b(parts3, x, w1):
    return pl.pallas_call(
        _tc_b_body,
        out_shape=(jax.ShapeDtypeStruct((N2 // 128, 128), jnp.float32),
                   jax.ShapeDtypeStruct((N, D), jnp.float32)),
    )(parts3, x, w1)


def _tc_d_body(d_ref, p_ref, xw_ref, b_ref, a_ref, w2_ref, out_ref):
    dv = d_ref[...]
    z = dv * (p_ref[0] + p_ref[1]) + dv * dv * xw_ref[...] + b_ref[...]
    a = a_ref[0, 0]
    z = jnp.where(z >= 0, z, a * z)
    out_ref[...] = jnp.dot(z, w2_ref[...], preferred_element_type=jnp.float32)


def _tc_d(dmat, p, xw, b, a, w2):
    return pl.pallas_call(
        _tc_d_body,
        out_shape=jax.ShapeDtypeStruct((N, D), jnp.float32),
    )(dmat, p, xw, b, a, w2)


def _tc_f_body(d_ref, p_ref, xw_ref, b_ref, a_ref, g_ref, be_ref, out_ref):
    dv = d_ref[...]
    z = dv * (p_ref[0] + p_ref[1]) + dv * dv * xw_ref[...] + b_ref[...]
    a = a_ref[0, 0]
    z = jnp.where(z >= 0, z, a * z)
    mu = jnp.mean(z, axis=0, keepdims=True)
    var = jnp.mean((z - mu) * (z - mu), axis=0, keepdims=True)
    h = (z - mu) * lax.rsqrt(var + 1e-5) * g_ref[...] + be_ref[...]
    out_ref[:, 0:D] = h
    out_ref[:, D:2 * D] = h


def _tc_f(dmat, p, xw, b, a, gamma, beta):
    return pl.pallas_call(
        _tc_f_body,
        out_shape=jax.ShapeDtypeStruct((N, 2 * D), jnp.float32),
    )(dmat, p, xw, b, a, gamma, beta)


def kernel(x, edge_weight, W1, b1, prelu_a, W2, b2, gamma, beta, edge_index):
    src = edge_index[0]
    dst = edge_index[1]
    pad = E_PAD - E
    zi = jnp.zeros((pad,), jnp.int32)
    srcp = jnp.concatenate([src, zi])
    dstp = jnp.concatenate([dst, zi])
    ewp = jnp.concatenate([edge_weight, jnp.zeros((pad,), jnp.float32)])

    parts = _deg_kernel(dstp, ewp)
    parts3 = parts.reshape(NW, N2 // 128, 128)
    dinv2d, xw1 = _tc_b(parts3, x, W1)
    dinv_flat = dinv2d.reshape(N2)

    p1, dfull = _msg_kernel_c1(xw1, dinv_flat, srcp, dstp, ewp)
    dmat = dfull[:N]

    a2 = prelu_a.reshape(1, 1)
    xw2 = _tc_d(dmat, p1, xw1, b1.reshape(1, D), a2, W2)

    p2 = _msg_kernel_c2(xw2, dinv_flat, srcp, dstp, ewp)
    out = _tc_f(dmat, p2, xw2, b2.reshape(1, D), a2,
                gamma.reshape(1, D), beta.reshape(1, D))
    return out


# speedup vs baseline: 8.1502x; 8.1502x over previous
import functools
import jax
import jax.numpy as jnp
from jax import lax
from jax.experimental import pallas as pl
from jax.experimental.pallas import tpu as pltpu
from jax.experimental.pallas import tpu_sc as plsc

N = 10000
D = 128
E = 320000
NC = 2
NS = 16
NW = NC * NS
CH = 128
EPT = 10112
E_PAD = EPT * NW
NCH = EPT // CH
N2 = 10240
RPT = N2 // NS
RQ = 128

_mesh = plsc.VectorSubcoreMesh(core_axis_name="c", subcore_axis_name="s")
_sc_params = pltpu.CompilerParams(needs_layout_passes=False)


@functools.partial(
    pl.kernel,
    out_type=jax.ShapeDtypeStruct((NW * N2,), jnp.float32),
    mesh=_mesh,
    compiler_params=_sc_params,
    scratch_types=[
        pltpu.VMEM((N2,), jnp.float32),
        pltpu.VMEM((CH,), jnp.int32),
        pltpu.VMEM((CH,), jnp.float32),
    ],
)
def _deg_kernel(dst_hbm, ew_hbm, parts_hbm, degv, dstv, ewv):
    cid = lax.axis_index("c")
    sid = lax.axis_index("s")
    wid = sid * NC + cid

    def zero(i, _):
        degv[pl.ds(i * 16, 16)] = jnp.zeros((16,), jnp.float32)
        return 0

    lax.fori_loop(0, N2 // 16, zero, 0)

    def chunk(j, _):
        base = wid * EPT + j * CH
        pltpu.sync_copy(dst_hbm.at[pl.ds(base, CH)], dstv)
        pltpu.sync_copy(ew_hbm.at[pl.ds(base, CH)], ewv)
        for g in range(CH // 16):
            idx = dstv[pl.ds(g * 16, 16)]
            w = ewv[pl.ds(g * 16, 16)]
            plsc.addupdate_scatter(degv, [idx], w)
        return 0

    lax.fori_loop(0, NCH, chunk, 0)
    pltpu.sync_copy(degv, parts_hbm.at[pl.ds(wid * N2, N2)])


def _msg_body(write_dfull, xw_hbm, dinv_hbm, src_hbm, dst_hbm, ew_hbm,
              p_hbm, dfull_hbm, acc, srcv, dstv, ewv, wbuf, rows, dinv_v,
              zbuf, sem):
    cid = lax.axis_index("c")
    sid = lax.axis_index("s")
    wid = sid * NC + cid

    pltpu.sync_copy(dinv_hbm, dinv_v)

    def zb(i, _):
        for k in range(8):
            zbuf[i, pl.ds(k * 16, 16)] = jnp.zeros((16,), jnp.float32)
        return 0

    lax.fori_loop(0, RQ, zb, 0)
    for q in range(RPT // RQ):
        pltpu.sync_copy(zbuf, acc.at[pl.ds(sid * RPT + q * RQ, RQ)])

    if write_dfull:
        @pl.when(cid == 0)
        def _():
            for q in range(RPT // RQ):
                def fill(r, _):
                    ridx = jnp.full((16,), sid * RPT + q * RQ + r, jnp.int32)
                    dv = plsc.load_gather(dinv_v, [ridx])
                    for k in range(8):
                        rows[r, pl.ds(k * 16, 16)] = dv
                    return 0

                lax.fori_loop(0, RQ, fill, 0)
                pltpu.sync_copy(
                    rows, dfull_hbm.at[pl.ds(sid * RPT + q * RQ, RQ)])

    plsc.subcore_barrier()

    def chunk(j, _):
        base = wid * EPT + j * CH
        pltpu.sync_copy(src_hbm.at[pl.ds(base, CH)], srcv)
        pltpu.sync_copy(dst_hbm.at[pl.ds(base, CH)], dstv)
        pltpu.sync_copy(ew_hbm.at[pl.ds(base, CH)], ewv)
        pltpu.async_copy(xw_hbm.at[srcv], rows, sem).wait()
        for g in range(CH // 16):
            idx = srcv[pl.ds(g * 16, 16)]
            dsv = plsc.load_gather(dinv_v, [idx])
            wbuf[pl.ds(g * 16, 16)] = ewv[pl.ds(g * 16, 16)] * dsv

        def srow(e, _):
            wv = plsc.load_gather(wbuf, [jnp.full((16,), e, jnp.int32)])
            for k in range(8):
                rows[e, pl.ds(k * 16, 16)] = rows[e, pl.ds(k * 16, 16)] * wv
            return 0

        lax.fori_loop(0, CH, srow, 0)
        pltpu.sync_copy(rows, acc.at[dstv], add=True)
        return 0

    lax.fori_loop(0, NCH, chunk, 0)
    plsc.subcore_barrier()

    for q in range(RPT // RQ):
        r0 = sid * RPT + q * RQ
        pltpu.sync_copy(acc.at[pl.ds(r0, RQ)], p_hbm.at[cid, pl.ds(r0, RQ)])


_msg_scratch = [
    pltpu.VMEM_SHARED((N2, D), jnp.float32),
    pltpu.VMEM((CH,), jnp.int32),
    pltpu.VMEM((CH,), jnp.int32),
    pltpu.VMEM((CH,), jnp.float32),
    pltpu.VMEM((CH,), jnp.float32),
    pltpu.VMEM((CH, D), jnp.float32),
    pltpu.VMEM((N2,), jnp.float32),
    pltpu.VMEM((RQ, D), jnp.float32),
    pltpu.SemaphoreType.DMA,
]

_msg_kernel_c1 = functools.partial(
    pl.kernel,
    out_type=(jax.ShapeDtypeStruct((NC, N2, D), jnp.float32),
              jax.ShapeDtypeStruct((N2, D), jnp.float32)),
    mesh=_mesh,
    compiler_params=_sc_params,
    scratch_types=_msg_scratch,
)(functools.partial(_msg_body, True))


def _msg_body2(xw_hbm, dinv_hbm, src_hbm, dst_hbm, ew_hbm, p_hbm,
               acc, srcv, dstv, ewv, wbuf, rows, dinv_v, zbuf, sem):
    _msg_body(False, xw_hbm, dinv_hbm, src_hbm, dst_hbm, ew_hbm,
              p_hbm, None, acc, srcv, dstv, ewv, wbuf, rows, dinv_v,
              zbuf, sem)


_msg_kernel_c2 = functools.partial(
    pl.kernel,
    out_type=jax.ShapeDtypeStruct((NC, N2, D), jnp.float32),
    mesh=_mesh,
    compiler_params=_sc_params,
    scratch_types=_msg_scratch,
)(_msg_body2)


def _tc_b_body(parts_ref, x_ref, w1_ref, dinv_ref, xw_ref):
    deg = 1.0 + jnp.sum(parts_ref[...], axis=0)
    dinv_ref[...] = lax.rsqrt(deg)
    xw_ref[...] = jnp.dot(x_ref[...], w1_ref[...],
                          preferred_element_type=jnp.float32)


def _tc_b(parts3, x, w1):
    return pl.pallas_call(
        _tc_b_body,
        out_shape=(jax.ShapeDtypeStruct((N2 // 128, 128), jnp.float32),
                   jax.ShapeDtypeStruct((N2, D), jnp.float32)),
    )(parts3, x, w1)


def _tc_d_body(d_ref, p_ref, xw_ref, b_ref, a_ref, w2_ref, out_ref):
    dv = d_ref[...]
    z = dv * (p_ref[0] + p_ref[1]) + dv * dv * xw_ref[...] + b_ref[...]
    a = a_ref[0, 0]
    z = jnp.where(z >= 0, z, a * z)
    out_ref[...] = jnp.dot(z, w2_ref[...], preferred_element_type=jnp.float32)


def _tc_d(dmat, p, xw, b, a, w2):
    return pl.pallas_call(
        _tc_d_body,
        out_shape=jax.ShapeDtypeStruct((N2, D), jnp.float32),
    )(dmat, p, xw, b, a, w2)


def _tc_f_body(d_ref, p_ref, xw_ref, b_ref, a_ref, g_ref, be_ref, out_ref):
    dv = d_ref[...]
    z = dv * (p_ref[0] + p_ref[1]) + dv * dv * xw_ref[...] + b_ref[...]
    a = a_ref[0, 0]
    z = jnp.where(z >= 0, z, a * z)
    z = z[:N]
    mu = jnp.mean(z, axis=0, keepdims=True)
    var = jnp.mean((z - mu) * (z - mu), axis=0, keepdims=True)
    h = (z - mu) * lax.rsqrt(var + 1e-5) * g_ref[...] + be_ref[...]
    out_ref[:, 0:D] = h
    out_ref[:, D:2 * D] = h


def _tc_f(dmat, p, xw, b, a, gamma, beta):
    return pl.pallas_call(
        _tc_f_body,
        out_shape=jax.ShapeDtypeStruct((N, 2 * D), jnp.float32),
    )(dmat, p, xw, b, a, gamma, beta)


def kernel(x, edge_weight, W1, b1, prelu_a, W2, b2, gamma, beta, edge_index):
    src = edge_index[0]
    dst = edge_index[1]
    pad = E_PAD - E
    zi = jnp.zeros((pad,), jnp.int32)
    srcp = jnp.concatenate([src, zi])
    dstp = jnp.concatenate([dst, zi])
    ewp = jnp.concatenate([edge_weight, jnp.zeros((pad,), jnp.float32)])

    xp = jnp.concatenate([x, jnp.zeros((N2 - N, D), jnp.float32)])

    parts = _deg_kernel(dstp, ewp)
    parts3 = parts.reshape(NW, N2 // 128, 128)
    dinv2d, xw1 = _tc_b(parts3, xp, W1)
    dinv_flat = dinv2d.reshape(N2)

    p1, dfull = _msg_kernel_c1(xw1, dinv_flat, srcp, dstp, ewp)

    a2 = prelu_a.reshape(1, 1)
    xw2 = _tc_d(dfull, p1, xw1, b1.reshape(1, D), a2, W2)

    p2 = _msg_kernel_c2(xw2, dinv_flat, srcp, dstp, ewp)
    out = _tc_f(dfull, p2, xw2, b2.reshape(1, D), a2,
                gamma.reshape(1, D), beta.reshape(1, D))
    return out
